# baseline (speedup 1.0000x reference)
import jax
import jax.numpy as jnp
from jax.experimental import pallas as pl
from jax.experimental.pallas import tpu as pltpu

_WINDOW = 8


def _sel_kernel(mask_ref, sel_ref, nsel_ref):
    c = mask_ref.shape[-1]
    nz = mask_ref[...] != 0.0
    nzi = nz.astype(jnp.int32)
    row = jax.lax.broadcasted_iota(jnp.int32, (c, c), 0)
    col = jax.lax.broadcasted_iota(jnp.int32, (c, c), 1)
    rank = jnp.sum((nz & (col < row)).astype(jnp.int32), axis=1)
    m = nz & (jnp.broadcast_to(rank[None, :], (c, c)) == row)
    sel = jnp.sum(jnp.where(m, col, 0), axis=1)
    sel_ref[...] = sel.reshape(1, c)
    nsel_ref[...] = jnp.sum(nzi, axis=-1, keepdims=True)


def _dma_gather_kernel(sel_ref, nsel_ref, in_hbm, out_hbm, nan_buf, sem):
    c = out_hbm.shape[1]
    nsel = nsel_ref[0]
    nan_buf[...] = jnp.full_like(nan_buf, jnp.nan)

    def start_copy(j):
        @pl.when(j < nsel)
        def _valid():
            pltpu.make_async_copy(
                in_hbm.at[:, pl.ds(sel_ref[j], 1)],
                out_hbm.at[:, pl.ds(j, 1)],
                sem,
            ).start()

        @pl.when(j >= nsel)
        def _invalid():
            pltpu.make_async_copy(
                nan_buf, out_hbm.at[:, pl.ds(j, 1)], sem
            ).start()

    def wait_one():
        pltpu.make_async_copy(
            in_hbm.at[:, pl.ds(0, 1)], out_hbm.at[:, pl.ds(0, 1)], sem
        ).wait()

    def body(j, carry):
        start_copy(j)

        @pl.when(j >= _WINDOW)
        def _():
            wait_one()

        return carry

    jax.lax.fori_loop(0, c, body, 0, unroll=False)

    def drain(j, carry):
        wait_one()
        return carry

    jax.lax.fori_loop(0, min(_WINDOW, c), drain, 0, unroll=False)


def kernel(input_tensor, indexes):
    n, c, h, w = input_tensor.shape

    sel, nsel = pl.pallas_call(
        _sel_kernel,
        out_shape=(
            jax.ShapeDtypeStruct((1, c), jnp.int32),
            jax.ShapeDtypeStruct((1, 1), jnp.int32),
        ),
    )(indexes.reshape(1, c))
    sel = sel.reshape(c)
    nsel = nsel.reshape(1)

    grid_spec = pltpu.PrefetchScalarGridSpec(
        num_scalar_prefetch=2,
        grid=(1,),
        in_specs=[pl.BlockSpec(memory_space=pltpu.MemorySpace.HBM)],
        out_specs=pl.BlockSpec(memory_space=pltpu.MemorySpace.HBM),
        scratch_shapes=[
            pltpu.VMEM((n, 1, h, w), input_tensor.dtype),
            pltpu.SemaphoreType.DMA,
        ],
    )
    return pl.pallas_call(
        _dma_gather_kernel,
        grid_spec=grid_spec,
        out_shape=jax.ShapeDtypeStruct((n, c, h, w), input_tensor.dtype),
    )(sel, nsel, input_tensor)


# speedup vs baseline: 40.8250x; 40.8250x over previous
import jax
import jax.numpy as jnp
from jax.experimental import pallas as pl
from jax.experimental.pallas import tpu as pltpu


def _sel_kernel(mask_ref, sel_ref, nsel_ref):
    c = mask_ref.shape[-1]
    nz = mask_ref[...] != 0.0
    nzi = nz.astype(jnp.int32)
    row = jax.lax.broadcasted_iota(jnp.int32, (c, c), 0)
    col = jax.lax.broadcasted_iota(jnp.int32, (c, c), 1)
    rank = jnp.sum((nz & (col < row)).astype(jnp.int32), axis=1)
    m = nz & (jnp.broadcast_to(rank[None, :], (c, c)) == row)
    sel = jnp.sum(jnp.where(m, col, 0), axis=1)
    sel_ref[...] = sel.reshape(1, c)
    nsel_ref[...] = jnp.sum(nzi, axis=-1, keepdims=True)


def _copy_kernel(sel_ref, nsel_ref, in_ref, out_ref):
    del sel_ref
    j = pl.program_id(0)

    @pl.when(j < nsel_ref[0])
    def _valid():
        out_ref[...] = in_ref[...]

    @pl.when(j >= nsel_ref[0])
    def _invalid():
        out_ref[...] = jnp.full_like(out_ref, jnp.nan)


def kernel(input_tensor, indexes):
    n, c, h, w = input_tensor.shape

    sel, nsel = pl.pallas_call(
        _sel_kernel,
        out_shape=(
            jax.ShapeDtypeStruct((1, c), jnp.int32),
            jax.ShapeDtypeStruct((1, 1), jnp.int32),
        ),
    )(indexes.reshape(1, c))
    sel = sel.reshape(c)
    nsel = nsel.reshape(1)

    grid_spec = pltpu.PrefetchScalarGridSpec(
        num_scalar_prefetch=2,
        grid=(c,),
        in_specs=[
            pl.BlockSpec(
                (n, 1, h, w), lambda j, sel_ref, nsel_ref: (0, sel_ref[j], 0, 0)
            )
        ],
        out_specs=pl.BlockSpec(
            (n, 1, h, w), lambda j, sel_ref, nsel_ref: (0, j, 0, 0)
        ),
    )
    return pl.pallas_call(
        _copy_kernel,
        grid_spec=grid_spec,
        out_shape=jax.ShapeDtypeStruct((n, c, h, w), input_tensor.dtype),
        compiler_params=pltpu.CompilerParams(
            dimension_semantics=("parallel",),
        ),
    )(sel, nsel, input_tensor)
